# baseline (speedup 1.0000x reference)
import functools

import jax
import jax.numpy as jnp
from jax import lax
from jax.experimental import pallas as pl
from jax.experimental.pallas import tpu as pltpu
from jax.experimental.pallas import tpu_sc as plsc

_NC, _NS = 2, 16
_NW = _NC * _NS
_CH = 128
_BLK_C = 2000
_BLK_N = 400


def _gelu(x):
    return 0.5 * x * (1.0 + lax.erf(x * (2.0 ** -0.5)))


def _ln_rows(x, g, b, eps=1e-5):
    mu = jnp.mean(x, axis=-1, keepdims=True)
    var = jnp.mean((x - mu) ** 2, axis=-1, keepdims=True)
    return (x - mu) * lax.rsqrt(var + eps) * g + b


def _make_gather(Ep, K, H):
    mesh = plsc.VectorSubcoreMesh(
        core_axis_name="c", subcore_axis_name="s",
        num_cores=_NC, num_subcores=_NS)
    Th = K // 2

    @functools.partial(
        pl.kernel,
        out_type=(jax.ShapeDtypeStruct((Ep, H), jnp.float32),
                  jax.ShapeDtypeStruct((Ep, H), jnp.float32)),
        mesh=mesh,
        scratch_types=[
            pltpu.VMEM((K, _CH), jnp.int32),
            pltpu.VMEM((K, _CH), jnp.int32),
            pltpu.VMEM((_CH, H), jnp.float32),
            pltpu.VMEM((_CH, H), jnp.float32),
            pltpu.VMEM((_CH, H), jnp.float32),
            pltpu.VMEM((_CH, H), jnp.float32),
            pltpu.SemaphoreType.DMA,
            pltpu.SemaphoreType.DMA,
            pltpu.SemaphoreType.DMA,
            pltpu.SemaphoreType.DMA,
        ],
    )
    def gather_k(h_hbm, src_hbm, dst_hbm, hs_hbm, hd_hbm,
                 idxs_v, idxd_v, bs0, bd0, bs1, bd1, sg0, sg1, so0, so1):
        c = lax.axis_index("c")
        s = lax.axis_index("s")
        w = s * _NC + c

        pltpu.sync_copy(src_hbm.at[pl.ds(w * K, K)], idxs_v)
        pltpu.sync_copy(dst_hbm.at[pl.ds(w * K, K)], idxd_v)

        def gstart(ci, bs, bd, sg):
            pltpu.async_copy(h_hbm.at[idxs_v.at[ci]], bs, sg)
            pltpu.async_copy(h_hbm.at[idxd_v.at[ci]], bd, sg)

        def gdrain(bs, bd, sg):
            pltpu.make_async_copy(h_hbm.at[pl.ds(0, _CH)], bs, sg).wait()
            pltpu.make_async_copy(h_hbm.at[pl.ds(0, _CH)], bd, sg).wait()

        def ostart(ci, bs, bd, so):
            base = (w * K + ci) * _CH
            pltpu.async_copy(bs, hs_hbm.at[pl.ds(base, _CH)], so)
            pltpu.async_copy(bd, hd_hbm.at[pl.ds(base, _CH)], so)

        def odrain(bs, bd, so):
            pltpu.make_async_copy(bs, hs_hbm.at[pl.ds(0, _CH)], so).wait()
            pltpu.make_async_copy(bd, hd_hbm.at[pl.ds(0, _CH)], so).wait()

        gstart(0, bs0, bd0, sg0)

        def body(t, carry):
            c0 = 2 * t
            c1 = c0 + 1
            gstart(c1, bs1, bd1, sg1)
            gdrain(bs0, bd0, sg0)
            ostart(c0, bs0, bd0, so0)
            gdrain(bs1, bd1, sg1)
            ostart(c1, bs1, bd1, so1)
            odrain(bs0, bd0, so0)

            @pl.when(t + 1 < Th)
            def _():
                gstart(c0 + 2, bs0, bd0, sg0)

            odrain(bs1, bd1, so1)
            return carry

        lax.fori_loop(0, Th, body, 0)

    return gather_k


def _make_counts(K, H, NR):
    mesh = plsc.VectorSubcoreMesh(
        core_axis_name="c", subcore_axis_name="s",
        num_cores=_NC, num_subcores=_NS)
    KB = 8
    Tb = K // KB

    @functools.partial(
        pl.kernel,
        out_type=jax.ShapeDtypeStruct((_NC, NR, H), jnp.float32),
        mesh=mesh,
        scratch_types=[
            pltpu.VMEM((K, _CH), jnp.int32),
            pltpu.VMEM((_CH, H), jnp.float32),
            pltpu.VMEM_SHARED((NR, H), jnp.float32),
            pltpu.SemaphoreType.DMA,
        ],
    )
    def counts_k(dst_hbm, ones_hbm, zeros_hbm, cnt_hbm,
                 idxd_v, ones_v, cacc, sem):
        c = lax.axis_index("c")
        s = lax.axis_index("s")
        w = s * _NC + c

        @pl.when(s == 0)
        def _():
            pltpu.sync_copy(zeros_hbm, cacc)

        pltpu.sync_copy(dst_hbm.at[pl.ds(w * K, K)], idxd_v)
        pltpu.sync_copy(ones_hbm, ones_v)
        plsc.subcore_barrier()

        def body(t, carry):
            for j in range(KB):
                pltpu.async_copy(ones_v, cacc.at[idxd_v.at[t * KB + j]],
                                 sem, add=True)
            for j in range(KB):
                pltpu.make_async_copy(
                    ones_v, cacc.at[idxd_v.at[0]], sem).wait()
            return carry

        lax.fori_loop(0, Tb, body, 0)
        plsc.subcore_barrier()

        @pl.when(s == 0)
        def _():
            pltpu.sync_copy(cacc, cnt_hbm.at[c])

    return counts_k


def _edge_body(hs_ref, hd_ref, e_ref, w1h_ref, w1e_ref, w2_ref,
               wvh_ref, wve_ref, be1_ref, be2_ref, bv1_ref, ge_ref, bbe_ref,
               g_ref, en_ref):
    hcat = jnp.concatenate([hs_ref[...], hd_ref[...]], axis=1)
    e = e_ref[...]
    pre1 = (jnp.dot(hcat, w1h_ref[...], preferred_element_type=jnp.float32)
            + jnp.dot(e, w1e_ref[...], preferred_element_type=jnp.float32)
            + be1_ref[...])
    t = jnp.dot(_gelu(pre1), w2_ref[...],
                preferred_element_type=jnp.float32) + be2_ref[...]
    en = _ln_rows(e + t, ge_ref[...], bbe_ref[...])
    pre2 = (jnp.dot(hcat, wvh_ref[...], preferred_element_type=jnp.float32)
            + jnp.dot(en, wve_ref[...], preferred_element_type=jnp.float32)
            + bv1_ref[...])
    g_ref[...] = _gelu(pre2)
    en_ref[...] = en


def _edge_mlp(HS, HD, e, W1h, W1e, W2, Wvh, Wve, be1, be2, bv1, g_e, b_e):
    Ep, H = HS.shape
    E, ED = e.shape
    H2 = 2 * H
    grid = (E // _BLK_C,)
    return pl.pallas_call(
        _edge_body,
        grid=grid,
        in_specs=[
            pl.BlockSpec((_BLK_C, H), lambda i: (i, 0)),
            pl.BlockSpec((_BLK_C, H), lambda i: (i, 0)),
            pl.BlockSpec((_BLK_C, ED), lambda i: (i, 0)),
            pl.BlockSpec((H2, ED), lambda i: (0, 0)),
            pl.BlockSpec((ED, ED), lambda i: (0, 0)),
            pl.BlockSpec((ED, ED), lambda i: (0, 0)),
            pl.BlockSpec((H2, H), lambda i: (0, 0)),
            pl.BlockSpec((ED, H), lambda i: (0, 0)),
            pl.BlockSpec((1, ED), lambda i: (0, 0)),
            pl.BlockSpec((1, ED), lambda i: (0, 0)),
            pl.BlockSpec((1, H), lambda i: (0, 0)),
            pl.BlockSpec((1, ED), lambda i: (0, 0)),
            pl.BlockSpec((1, ED), lambda i: (0, 0)),
        ],
        out_specs=[
            pl.BlockSpec((_BLK_C, H), lambda i: (i, 0)),
            pl.BlockSpec((_BLK_C, ED), lambda i: (i, 0)),
        ],
        out_shape=[
            jax.ShapeDtypeStruct((Ep, H), jnp.float32),
            jax.ShapeDtypeStruct((E, ED), jnp.float32),
        ],
    )(HS, HD, e, W1h, W1e, W2, Wvh, Wve, be1, be2, bv1, g_e, b_e)


def _make_scatter(Ep, K, H, NR):
    mesh = plsc.VectorSubcoreMesh(
        core_axis_name="c", subcore_axis_name="s",
        num_cores=_NC, num_subcores=_NS)
    Th = K // 2

    @functools.partial(
        pl.kernel,
        out_type=jax.ShapeDtypeStruct((_NC, NR, H), jnp.float32),
        mesh=mesh,
        scratch_types=[
            pltpu.VMEM((K, _CH), jnp.int32),
            pltpu.VMEM((_CH, H), jnp.float32),
            pltpu.VMEM((_CH, H), jnp.float32),
            pltpu.VMEM_SHARED((NR, H), jnp.float32),
            pltpu.SemaphoreType.DMA,
            pltpu.SemaphoreType.DMA,
        ],
    )
    def scatter_k(g_hbm, dst_hbm, zeros_hbm, out_hbm,
                  idxd_v, gbuf0, gbuf1, acc, si0, si1):
        c = lax.axis_index("c")
        s = lax.axis_index("s")
        w = s * _NC + c

        @pl.when(s == 0)
        def _():
            pltpu.sync_copy(zeros_hbm, acc)

        pltpu.sync_copy(dst_hbm.at[pl.ds(w * K, K)], idxd_v)
        plsc.subcore_barrier()

        def istart(ci, gbuf, si):
            base = (w * K + ci) * _CH
            pltpu.async_copy(g_hbm.at[pl.ds(base, _CH)], gbuf, si)

        def idrain(gbuf, si):
            pltpu.make_async_copy(g_hbm.at[pl.ds(0, _CH)], gbuf, si).wait()

        istart(0, gbuf0, si0)

        def body(t, carry):
            c0 = 2 * t
            c1 = c0 + 1
            istart(c1, gbuf1, si1)
            idrain(gbuf0, si0)
            pltpu.sync_copy(gbuf0, acc.at[idxd_v.at[c0]], add=True)

            @pl.when(t + 1 < Th)
            def _():
                istart(c0 + 2, gbuf0, si0)

            idrain(gbuf1, si1)
            pltpu.sync_copy(gbuf1, acc.at[idxd_v.at[c1]], add=True)
            return carry

        lax.fori_loop(0, Th, body, 0)
        plsc.subcore_barrier()

        @pl.when(s == 0)
        def _():
            pltpu.sync_copy(acc, out_hbm.at[c])

    return scatter_k


def _node_body(acc_ref, cnt_ref, h_ref, wv2_ref, bv2_ref, gv_ref, bbv_ref,
               out_ref):
    a = acc_ref[0] + acc_ref[1]
    cnt = cnt_ref[0][:, :1] + cnt_ref[1][:, :1]
    aggp = a / jnp.maximum(cnt, 1.0)
    agg = jnp.dot(aggp, wv2_ref[...], preferred_element_type=jnp.float32) \
        + bv2_ref[...] * (cnt > 0.0).astype(jnp.float32)
    out_ref[...] = _ln_rows(h_ref[...] + agg, gv_ref[...], bbv_ref[...])


def _node_update(ACC, CNT, h, Wv2T, bv2, g_v, b_v):
    N, H = h.shape
    grid = (N // _BLK_N,)
    return pl.pallas_call(
        _node_body,
        grid=grid,
        in_specs=[
            pl.BlockSpec((_NC, _BLK_N, H), lambda i: (0, i, 0)),
            pl.BlockSpec((_NC, _BLK_N, H), lambda i: (0, i, 0)),
            pl.BlockSpec((_BLK_N, H), lambda i: (i, 0)),
            pl.BlockSpec((H, H), lambda i: (0, 0)),
            pl.BlockSpec((1, H), lambda i: (0, 0)),
            pl.BlockSpec((1, H), lambda i: (0, 0)),
            pl.BlockSpec((1, H), lambda i: (0, 0)),
        ],
        out_specs=pl.BlockSpec((_BLK_N, H), lambda i: (i, 0)),
        out_shape=jax.ShapeDtypeStruct((N, H), jnp.float32),
    )(ACC, CNT, h, Wv2T, bv2, g_v, b_v)


def kernel(h, e, edge_index, We1, be1, We2, be2, Wv1, bv1, Wv2, bv2,
           g_e, b_e, g_v, b_v):
    N, H = h.shape
    E, ED = e.shape

    W1h = We1[:, :2 * H].T
    W1e = We1[:, 2 * H:].T
    W2 = We2.T
    Wvh = Wv1[:, :2 * H].T
    Wve = Wv1[:, 2 * H:].T
    Wv2T = Wv2.T

    K = -(-E // (_NW * _CH))
    K = K + (-K) % 8
    Ep = _NW * K * _CH
    pad = Ep - E
    NR = N + 8
    src_p = jnp.concatenate(
        [edge_index[0], jnp.zeros((pad,), jnp.int32)]).reshape(_NW * K, _CH)
    dstg_p = jnp.concatenate(
        [edge_index[1], jnp.zeros((pad,), jnp.int32)]).reshape(_NW * K, _CH)
    dsts_p = jnp.concatenate(
        [edge_index[1], jnp.full((pad,), N, jnp.int32)]).reshape(_NW * K, _CH)
    zeros = jnp.zeros((NR, H), jnp.float32)
    ones = jnp.ones((_CH, H), jnp.float32)

    r2 = lambda v: v.reshape(1, -1)

    HS, HD = _make_gather(Ep, K, H)(h, src_p, dstg_p)
    CNT = _make_counts(K, H, NR)(dsts_p, ones, zeros)
    G, EN = _edge_mlp(HS, HD, e, W1h, W1e, W2, Wvh, Wve,
                      r2(be1), r2(be2), r2(bv1), r2(g_e), r2(b_e))
    ACC = _make_scatter(Ep, K, H, NR)(G, dsts_p, zeros)
    h_new = _node_update(ACC, CNT, h, Wv2T, r2(bv2), r2(g_v), r2(b_v))
    return (h_new, EN)


# speedup vs baseline: 2.1596x; 2.1596x over previous
import functools

import jax
import jax.numpy as jnp
from jax import lax
from jax.experimental import pallas as pl
from jax.experimental.pallas import tpu as pltpu
from jax.experimental.pallas import tpu_sc as plsc

_NC, _NS = 2, 16
_NW = _NC * _NS
_CH = 128
_BLK_C = 2000
_BLK_N = 400


def _gelu(x):
    return 0.5 * x * (1.0 + lax.erf(x * (2.0 ** -0.5)))


def _ln_rows(x, g, b, eps=1e-5):
    mu = jnp.mean(x, axis=-1, keepdims=True)
    var = jnp.mean((x - mu) ** 2, axis=-1, keepdims=True)
    return (x - mu) * lax.rsqrt(var + eps) * g + b


def _make_gather(Ep, K, H, N):
    mesh = plsc.VectorSubcoreMesh(
        core_axis_name="c", subcore_axis_name="s",
        num_cores=_NC, num_subcores=_NS)
    Th = K // 2

    @functools.partial(
        pl.kernel,
        out_type=(jax.ShapeDtypeStruct((Ep, H), jnp.float32),
                  jax.ShapeDtypeStruct((Ep, H), jnp.float32)),
        mesh=mesh,
        scratch_types=[
            pltpu.VMEM((K, _CH), jnp.int32),
            pltpu.VMEM((_CH, H), jnp.float32),
            pltpu.VMEM((_CH, H), jnp.float32),
            pltpu.VMEM_SHARED((N, H), jnp.float32),
            pltpu.SemaphoreType.DMA,
            pltpu.SemaphoreType.DMA,
            pltpu.SemaphoreType.DMA,
            pltpu.SemaphoreType.DMA,
        ],
    )
    def gather_k(h_hbm, src_hbm, dst_hbm, hs_hbm, hd_hbm,
                 idx_v, b0, b1, h_spm, sg0, sg1, so0, so1):
        c = lax.axis_index("c")
        s = lax.axis_index("s")
        w = s * _NC + c

        @pl.when(s == 0)
        def _():
            pltpu.sync_copy(h_hbm, h_spm)

        pltpu.sync_copy(src_hbm.at[pl.ds(w * K, K)], idx_v)
        plsc.subcore_barrier()

        def one_pass(out_hbm):
            def gstart(ci, b, sg):
                pltpu.async_copy(h_spm.at[idx_v.at[ci]], b, sg)

            def gdrain(b, sg):
                pltpu.make_async_copy(h_hbm.at[pl.ds(0, _CH)], b, sg).wait()

            def ostart(ci, b, so):
                base = (w * K + ci) * _CH
                pltpu.async_copy(b, out_hbm.at[pl.ds(base, _CH)], so)

            def odrain(b, so):
                pltpu.make_async_copy(b, out_hbm.at[pl.ds(0, _CH)], so).wait()

            gstart(0, b0, sg0)

            def body(t, carry):
                c0 = 2 * t
                c1 = c0 + 1
                gstart(c1, b1, sg1)
                gdrain(b0, sg0)
                ostart(c0, b0, so0)
                gdrain(b1, sg1)
                ostart(c1, b1, so1)
                odrain(b0, so0)

                @pl.when(t + 1 < Th)
                def _():
                    gstart(c0 + 2, b0, sg0)

                odrain(b1, so1)
                return carry

            lax.fori_loop(0, Th, body, 0)

        one_pass(hs_hbm)
        pltpu.sync_copy(dst_hbm.at[pl.ds(w * K, K)], idx_v)
        one_pass(hd_hbm)

    return gather_k


def _make_counts(K, H, NR):
    mesh = plsc.VectorSubcoreMesh(
        core_axis_name="c", subcore_axis_name="s",
        num_cores=_NC, num_subcores=_NS)
    KB = 8
    Tb = K // KB

    @functools.partial(
        pl.kernel,
        out_type=jax.ShapeDtypeStruct((_NC, NR, H), jnp.float32),
        mesh=mesh,
        scratch_types=[
            pltpu.VMEM((K, _CH), jnp.int32),
            pltpu.VMEM((_CH, H), jnp.float32),
            pltpu.VMEM_SHARED((NR, H), jnp.float32),
            pltpu.SemaphoreType.DMA,
        ],
    )
    def counts_k(dst_hbm, ones_hbm, zeros_hbm, cnt_hbm,
                 idxd_v, ones_v, cacc, sem):
        c = lax.axis_index("c")
        s = lax.axis_index("s")
        w = s * _NC + c

        @pl.when(s == 0)
        def _():
            pltpu.sync_copy(zeros_hbm, cacc)

        pltpu.sync_copy(dst_hbm.at[pl.ds(w * K, K)], idxd_v)
        pltpu.sync_copy(ones_hbm, ones_v)
        plsc.subcore_barrier()

        def body(t, carry):
            for j in range(KB):
                pltpu.async_copy(ones_v, cacc.at[idxd_v.at[t * KB + j]],
                                 sem, add=True)
            for j in range(KB):
                pltpu.make_async_copy(
                    ones_v, cacc.at[idxd_v.at[0]], sem).wait()
            return carry

        lax.fori_loop(0, Tb, body, 0)
        plsc.subcore_barrier()

        @pl.when(s == 0)
        def _():
            pltpu.sync_copy(cacc, cnt_hbm.at[c])

    return counts_k


def _edge_body(hs_ref, hd_ref, e_ref, w1h_ref, w1e_ref, w2_ref,
               wvh_ref, wve_ref, be1_ref, be2_ref, bv1_ref, ge_ref, bbe_ref,
               g_ref, en_ref):
    hcat = jnp.concatenate([hs_ref[...], hd_ref[...]],
                           axis=1).astype(jnp.bfloat16)
    e = e_ref[...]
    pre1 = (jnp.dot(hcat, w1h_ref[...], preferred_element_type=jnp.float32)
            + jnp.dot(e, w1e_ref[...], preferred_element_type=jnp.float32)
            + be1_ref[...])
    t = jnp.dot(_gelu(pre1), w2_ref[...],
                preferred_element_type=jnp.float32) + be2_ref[...]
    en = _ln_rows(e + t, ge_ref[...], bbe_ref[...])
    pre2 = (jnp.dot(hcat, wvh_ref[...], preferred_element_type=jnp.float32)
            + jnp.dot(en, wve_ref[...], preferred_element_type=jnp.float32)
            + bv1_ref[...])
    g_ref[...] = _gelu(pre2)
    en_ref[...] = en


def _edge_mlp(HS, HD, e, W1h, W1e, W2, Wvh, Wve, be1, be2, bv1, g_e, b_e):
    Ep, H = HS.shape
    E, ED = e.shape
    H2 = 2 * H
    grid = (E // _BLK_C,)
    return pl.pallas_call(
        _edge_body,
        grid=grid,
        in_specs=[
            pl.BlockSpec((_BLK_C, H), lambda i: (i, 0)),
            pl.BlockSpec((_BLK_C, H), lambda i: (i, 0)),
            pl.BlockSpec((_BLK_C, ED), lambda i: (i, 0)),
            pl.BlockSpec((H2, ED), lambda i: (0, 0)),
            pl.BlockSpec((ED, ED), lambda i: (0, 0)),
            pl.BlockSpec((ED, ED), lambda i: (0, 0)),
            pl.BlockSpec((H2, H), lambda i: (0, 0)),
            pl.BlockSpec((ED, H), lambda i: (0, 0)),
            pl.BlockSpec((1, ED), lambda i: (0, 0)),
            pl.BlockSpec((1, ED), lambda i: (0, 0)),
            pl.BlockSpec((1, H), lambda i: (0, 0)),
            pl.BlockSpec((1, ED), lambda i: (0, 0)),
            pl.BlockSpec((1, ED), lambda i: (0, 0)),
        ],
        out_specs=[
            pl.BlockSpec((_BLK_C, H), lambda i: (i, 0)),
            pl.BlockSpec((_BLK_C, ED), lambda i: (i, 0)),
        ],
        out_shape=[
            jax.ShapeDtypeStruct((Ep, H), jnp.float32),
            jax.ShapeDtypeStruct((E, ED), jnp.float32),
        ],
    )(HS, HD, e, W1h, W1e, W2, Wvh, Wve, be1, be2, bv1, g_e, b_e)


def _make_scatter(Ep, K, H, NR):
    mesh = plsc.VectorSubcoreMesh(
        core_axis_name="c", subcore_axis_name="s",
        num_cores=_NC, num_subcores=_NS)
    Th = K // 2

    @functools.partial(
        pl.kernel,
        out_type=jax.ShapeDtypeStruct((_NC, NR, H), jnp.float32),
        mesh=mesh,
        scratch_types=[
            pltpu.VMEM((K, _CH), jnp.int32),
            pltpu.VMEM((_CH, H), jnp.float32),
            pltpu.VMEM((_CH, H), jnp.float32),
            pltpu.VMEM_SHARED((NR, H), jnp.float32),
            pltpu.SemaphoreType.DMA,
            pltpu.SemaphoreType.DMA,
        ],
    )
    def scatter_k(g_hbm, dst_hbm, zeros_hbm, out_hbm,
                  idxd_v, gbuf0, gbuf1, acc, si0, si1):
        c = lax.axis_index("c")
        s = lax.axis_index("s")
        w = s * _NC + c

        @pl.when(s == 0)
        def _():
            pltpu.sync_copy(zeros_hbm, acc)

        pltpu.sync_copy(dst_hbm.at[pl.ds(w * K, K)], idxd_v)
        plsc.subcore_barrier()

        def istart(ci, gbuf, si):
            base = (w * K + ci) * _CH
            pltpu.async_copy(g_hbm.at[pl.ds(base, _CH)], gbuf, si)

        def idrain(gbuf, si):
            pltpu.make_async_copy(g_hbm.at[pl.ds(0, _CH)], gbuf, si).wait()

        istart(0, gbuf0, si0)

        def body(t, carry):
            c0 = 2 * t
            c1 = c0 + 1
            istart(c1, gbuf1, si1)
            idrain(gbuf0, si0)
            pltpu.sync_copy(gbuf0, acc.at[idxd_v.at[c0]], add=True)

            @pl.when(t + 1 < Th)
            def _():
                istart(c0 + 2, gbuf0, si0)

            idrain(gbuf1, si1)
            pltpu.sync_copy(gbuf1, acc.at[idxd_v.at[c1]], add=True)
            return carry

        lax.fori_loop(0, Th, body, 0)
        plsc.subcore_barrier()

        @pl.when(s == 0)
        def _():
            pltpu.sync_copy(acc, out_hbm.at[c])

    return scatter_k


def _node_body(acc_ref, cnt_ref, h_ref, wv2_ref, bv2_ref, gv_ref, bbv_ref,
               out_ref):
    a = acc_ref[0] + acc_ref[1]
    cnt = cnt_ref[0][:, :1] + cnt_ref[1][:, :1]
    aggp = a / jnp.maximum(cnt, 1.0)
    agg = jnp.dot(aggp, wv2_ref[...], preferred_element_type=jnp.float32) \
        + bv2_ref[...] * (cnt > 0.0).astype(jnp.float32)
    out_ref[...] = _ln_rows(h_ref[...] + agg, gv_ref[...], bbv_ref[...])


def _node_update(ACC, CNT, h, Wv2T, bv2, g_v, b_v):
    N, H = h.shape
    grid = (N // _BLK_N,)
    return pl.pallas_call(
        _node_body,
        grid=grid,
        in_specs=[
            pl.BlockSpec((_NC, _BLK_N, H), lambda i: (0, i, 0)),
            pl.BlockSpec((_NC, _BLK_N, H), lambda i: (0, i, 0)),
            pl.BlockSpec((_BLK_N, H), lambda i: (i, 0)),
            pl.BlockSpec((H, H), lambda i: (0, 0)),
            pl.BlockSpec((1, H), lambda i: (0, 0)),
            pl.BlockSpec((1, H), lambda i: (0, 0)),
            pl.BlockSpec((1, H), lambda i: (0, 0)),
        ],
        out_specs=pl.BlockSpec((_BLK_N, H), lambda i: (i, 0)),
        out_shape=jax.ShapeDtypeStruct((N, H), jnp.float32),
    )(ACC, CNT, h, Wv2T, bv2, g_v, b_v)


def kernel(h, e, edge_index, We1, be1, We2, be2, Wv1, bv1, Wv2, bv2,
           g_e, b_e, g_v, b_v):
    N, H = h.shape
    E, ED = e.shape

    W1h = We1[:, :2 * H].T
    W1e = We1[:, 2 * H:].T
    W2 = We2.T
    Wvh = Wv1[:, :2 * H].T
    Wve = Wv1[:, 2 * H:].T
    Wv2T = Wv2.T

    K = -(-E // (_NW * _CH))
    K = K + (-K) % 8
    Ep = _NW * K * _CH
    pad = Ep - E
    NR = N + 8
    src_p = jnp.concatenate(
        [edge_index[0], jnp.zeros((pad,), jnp.int32)]).reshape(_NW * K, _CH)
    dstg_p = jnp.concatenate(
        [edge_index[1], jnp.zeros((pad,), jnp.int32)]).reshape(_NW * K, _CH)
    dsts_p = jnp.concatenate(
        [edge_index[1], jnp.full((pad,), N, jnp.int32)]).reshape(_NW * K, _CH)
    zeros = jnp.zeros((NR, H), jnp.float32)
    ones = jnp.ones((_CH, H), jnp.float32)

    r2 = lambda v: v.reshape(1, -1)

    HS, HD = _make_gather(Ep, K, H, N)(h, src_p, dstg_p)
    CNT = _make_counts(K, H, NR)(dsts_p, ones, zeros)
    G, EN = _edge_mlp(HS, HD, e, W1h.astype(jnp.bfloat16), W1e, W2,
                      Wvh.astype(jnp.bfloat16), Wve,
                      r2(be1), r2(be2), r2(bv1), r2(g_e), r2(b_e))
    ACC = _make_scatter(Ep, K, H, NR)(G, dsts_p, zeros)
    h_new = _node_update(ACC, CNT, h, Wv2T, r2(bv2), r2(g_v), r2(b_v))
    return (h_new, EN)
